# Initial kernel scaffold; baseline (speedup 1.0000x reference)
import jax
import jax.numpy as jnp
from jax.experimental import pallas as pl


def kernel(feats, edge_index, W, b, prelu_a):
    raise NotImplementedError("write your pallas kernel here")


# speedup vs baseline: 7.5372x; 7.5372x over previous
import functools

import jax
import jax.numpy as jnp
from jax import lax
from jax.experimental import pallas as pl
from jax.experimental.pallas import tpu as pltpu
from jax.experimental.pallas import tpu_sc as plsc

N = 10000
E = 320000
D = 128
NP = 10240
EP = 323584
NB = 80
CH = EP // 32 // 128
RPT = NP // 16

_mesh = plsc.VectorSubcoreMesh(core_axis_name="c", subcore_axis_name="s",
                               num_cores=2, num_subcores=16)


def _fill1d(ref, n, val, dtype):
    def body(i, c):
        ref[pl.ds(i * 16, 16)] = jnp.full((16,), val, dtype)
        return c
    lax.fori_loop(0, n // 16, body, 0)


@functools.partial(
    pl.kernel,
    out_type=jax.ShapeDtypeStruct((4 * NP,), jnp.float32),
    mesh=_mesh,
    scratch_types=[
        pltpu.VMEM_SHARED((NP,), jnp.float32),
        pltpu.VMEM_SHARED((NP,), jnp.float32),
        pltpu.VMEM((CH, 128), jnp.int32),
        pltpu.VMEM((CH, 128), jnp.int32),
        pltpu.VMEM((128,), jnp.float32),
        pltpu.VMEM((RPT,), jnp.float32),
    ],
)
def _deg_kernel(src_hbm, dst_hbm, degp_hbm,
                dout_sp, din_sp, sidx, didx, ones_v, zbuf):
    cid = lax.axis_index("c")
    sid = lax.axis_index("s")
    wid = cid * 16 + sid
    pltpu.sync_copy(src_hbm.at[wid], sidx)
    pltpu.sync_copy(dst_hbm.at[wid], didx)

    _fill1d(zbuf, RPT, 0.0, jnp.float32)
    pltpu.sync_copy(zbuf, dout_sp.at[pl.ds(sid * RPT, RPT)])
    pltpu.sync_copy(zbuf, din_sp.at[pl.ds(sid * RPT, RPT)])
    _fill1d(ones_v, 128, 1.0, jnp.float32)
    plsc.subcore_barrier()

    def body(j, carry):
        pltpu.sync_copy(ones_v, dout_sp.at[sidx.at[j]], add=True)
        pltpu.sync_copy(ones_v, din_sp.at[didx.at[j]], add=True)
        return carry

    lax.fori_loop(0, CH, body, 0)
    plsc.subcore_barrier()
    base = cid * (2 * NP) + sid * RPT
    pltpu.sync_copy(dout_sp.at[pl.ds(sid * RPT, RPT)],
                    degp_hbm.at[pl.ds(base, RPT)])
    pltpu.sync_copy(din_sp.at[pl.ds(sid * RPT, RPT)],
                    degp_hbm.at[pl.ds(base + NP, RPT)])


def _prep_body(feats_ref, degp_ref, h_ref):
    i = pl.program_id(0)
    f = feats_ref[...]
    deg = degp_ref[...]
    deg_out = deg[0, 0, :] + deg[1, 0, :]
    norm = jnp.where(deg_out > 0, lax.rsqrt(deg_out), 0.0)
    row = i * 128 + lax.broadcasted_iota(jnp.int32, (128, 1), 0)
    h_ref[...] = jnp.where(row < N, f * norm[:, None], 0.0)


_prep_call = pl.pallas_call(
    _prep_body,
    grid=(NB,),
    in_specs=[
        pl.BlockSpec((128, D), lambda i: (i, 0)),
        pl.BlockSpec((2, 2, 128), lambda i: (0, 0, i)),
    ],
    out_specs=pl.BlockSpec((128, D), lambda i: (i, 0)),
    out_shape=jax.ShapeDtypeStruct((NP, D), jnp.float32),
)


@functools.partial(
    pl.kernel,
    out_type=jax.ShapeDtypeStruct((2, NP, D), jnp.float32),
    mesh=_mesh,
    scratch_types=[
        pltpu.VMEM_SHARED((NP, D), jnp.float32),
        pltpu.VMEM((CH, 128), jnp.int32),
        pltpu.VMEM((CH, 128), jnp.int32),
        pltpu.VMEM((128, D), jnp.float32),
        pltpu.SemaphoreType.DMA,
    ],
)
def _agg_kernel(h_hbm, srcb_hbm, dstb_hbm, aggp_hbm,
                agg_sp, sidx, didx, rows, sem):
    cid = lax.axis_index("c")
    sid = lax.axis_index("s")
    wid = cid * 16 + sid
    pltpu.sync_copy(srcb_hbm.at[wid], sidx)
    pltpu.sync_copy(dstb_hbm.at[wid], didx)

    def zbody(i, c):
        for k in range(D // 16):
            rows[i, pl.ds(k * 16, 16)] = jnp.zeros((16,), jnp.float32)
        return c
    lax.fori_loop(0, 128, zbody, 0)
    for c in range(RPT // 128):
        pltpu.sync_copy(rows, agg_sp.at[pl.ds(sid * RPT + c * 128, 128)])
    plsc.subcore_barrier()

    def body(j, carry):
        pltpu.async_copy(h_hbm.at[sidx.at[j]], rows, sem).wait()
        pltpu.sync_copy(rows, agg_sp.at[didx.at[j]], add=True)
        return carry

    lax.fori_loop(0, CH, body, 0)
    plsc.subcore_barrier()
    pltpu.sync_copy(agg_sp.at[pl.ds(sid * RPT, RPT)],
                    aggp_hbm.at[cid, pl.ds(sid * RPT, RPT)])


def _final_body(aggp_ref, degp_ref, w_ref, b_ref, a_ref, out_ref):
    agg = aggp_ref[0] + aggp_ref[1]
    deg = degp_ref[...]
    deg_in = deg[0, 1, :] + deg[1, 1, :]
    norm = jnp.where(deg_in > 0, lax.rsqrt(deg_in), 0.0)
    rst = agg * norm[:, None]
    o = jnp.dot(rst, w_ref[...], preferred_element_type=jnp.float32)
    o = o + b_ref[0, :]
    a = a_ref[0, 0]
    out_ref[...] = jnp.where(o >= 0, o, a * o)


_final_call = pl.pallas_call(
    _final_body,
    grid=(NB,),
    in_specs=[
        pl.BlockSpec((2, 128, D), lambda i: (0, i, 0)),
        pl.BlockSpec((2, 2, 128), lambda i: (0, 0, i)),
        pl.BlockSpec((D, D), lambda i: (0, 0)),
        pl.BlockSpec((1, D), lambda i: (0, 0)),
        pl.BlockSpec((1, 1), lambda i: (0, 0)),
    ],
    out_specs=pl.BlockSpec((128, D), lambda i: (i, 0)),
    out_shape=jax.ShapeDtypeStruct((NP, D), jnp.float32),
)


def kernel(feats, edge_index, W, b, prelu_a):
    src = edge_index[0].astype(jnp.int32)
    dst = edge_index[1].astype(jnp.int32)
    pad = N + (jnp.arange(EP - E, dtype=jnp.int32) % (NP - N))
    src_b = jnp.concatenate([src, pad]).reshape(32, CH, 128)
    dst_b = jnp.concatenate([dst, pad]).reshape(32, CH, 128)
    feats_pad = jnp.pad(feats, ((0, NP - N), (0, 0)))

    degp = _deg_kernel(src_b, dst_b).reshape(2, 2, NP)
    h = _prep_call(feats_pad, degp)
    aggp = _agg_kernel(h, src_b, dst_b)
    out = _final_call(aggp, degp, W, b.reshape(1, D),
                      prelu_a.reshape(1, 1))
    return out[:N]
